# baseline (speedup 1.0000x reference)
import math

import jax
import jax.numpy as jnp
from jax import lax
from jax.experimental import pallas as pl
from jax.experimental.pallas import tpu as pltpu
from jax.experimental.pallas import tpu_sc as plsc

def _cp(tc_tiling):
    cp = pltpu.CompilerParams(use_tc_tiling_on_sc=tc_tiling)
    if "needs_layout_passes" in pltpu.CompilerParams.__dataclass_fields__:
        import dataclasses
        cp = dataclasses.replace(cp, needs_layout_passes=False)
    return cp


NC = 2
NS = 16
NW = NC * NS
V = 1000000
D = 64
SCALE = math.sqrt(D)

K0_CHUNKS_MAIN = 7808
K0_PER_W = K0_CHUNKS_MAIN // NW


def _table_format(t_cm):
    mesh = plsc.VectorSubcoreMesh(core_axis_name="c", subcore_axis_name="s")

    @pl.kernel(
        out_type=jax.ShapeDtypeStruct((V // 2, 128), jnp.float32),
        mesh=mesh,
        compiler_params=_cp(True),
        scratch_types=[
            pltpu.VMEM((2, 64, 128), jnp.float32),
            pltpu.VMEM((2, 64, 128), jnp.float32),
            pltpu.SemaphoreType.DMA((2,)),
            pltpu.SemaphoreType.DMA((2,)),
        ],
    )
    def k0(tcm_hbm, out_hbm, vbuf, sbuf, gsem, ssem):
        wid = lax.axis_index("c") * NS + lax.axis_index("s")
        j0 = wid * K0_PER_W
        iota = lax.iota(jnp.int32, 16)

        def in_copies(j, b):
            return [
                pltpu.make_async_copy(
                    tcm_hbm.at[pl.ds(cg * 8, 8), pl.ds(j * 128, 128)],
                    vbuf.at[b, pl.ds(cg * 8, 8), :], gsem.at[b])
                for cg in range(8)
            ]

        def store_copy(j, b, rows=64):
            return pltpu.make_async_copy(
                sbuf.at[b, pl.ds(0, rows), :],
                out_hbm.at[pl.ds(j * 64, rows), :], ssem.at[b])

        def transpose_chunk(b, nrow=64):
            v = vbuf.at[b]
            s = sbuf.at[b]

            @pl.loop(0, nrow)
            def _(p):
                for h in range(2):
                    r = jnp.broadcast_to((2 * p + h).astype(jnp.int32), (16,))
                    for l0 in range(4):
                        c = iota + (l0 * 16)
                        vec = plsc.load_gather(v, [c, r])
                        s.at[p, pl.ds(h * 64 + l0 * 16, 16)][...] = vec

        def process(j, b, wait_store, issue_next):
            for cp in in_copies(j, b):
                cp.wait()
            if wait_store:
                store_copy(j, b).wait()
            transpose_chunk(b)
            if issue_next:
                for cp in in_copies(j + 2, b):
                    cp.start()
            store_copy(j, b).start()

        for b in range(2):
            for cp in in_copies(j0 + b, b):
                cp.start()
        for b in range(2):
            process(j0 + b, b, wait_store=False, issue_next=True)

        @pl.loop(1, K0_PER_W // 2 - 1)
        def _(g):
            jj = j0 + 2 * g
            for b in range(2):
                process(jj + b, b, wait_store=True, issue_next=True)

        jj = j0 + K0_PER_W - 2
        for b in range(2):
            process(jj + b, b, wait_store=True, issue_next=False)
        for b in range(2):
            store_copy(jj + b, b).wait()

        @pl.when(wid < 4)
        def _():
            j = K0_CHUNKS_MAIN + wid
            for cp in in_copies(j, 0):
                cp.start()
            for cp in in_copies(j, 0):
                cp.wait()
            transpose_chunk(0)
            store_copy(j, 0).start()
            store_copy(j, 0).wait()

    return k0(t_cm)


SB = 25
BB = 32


TAIL0 = K0_CHUNKS_MAIN * 128 + 4 * 128


def _gather_scale(tok6, tab_rows, tail):
    mesh = plsc.VectorSubcoreMesh(core_axis_name="c", subcore_axis_name="s")

    @pl.kernel(
        out_type=jax.ShapeDtypeStruct((200, 8, BB, 8, 128), jnp.float32),
        mesh=mesh,
        compiler_params=_cp(False),
        scratch_types=[
            pltpu.VMEM((SB, 8, 128), jnp.int32),
            pltpu.VMEM((SB, 8, 128), jnp.int32),
            pltpu.VMEM((2, 192, D), jnp.float32),
            pltpu.VMEM((2, 8, 8, 128), jnp.float32),
            pltpu.SemaphoreType.DMA((2,)),
            pltpu.SemaphoreType.DMA((2,)),
        ],
    )
    def k2(tok_hbm, tab_hbm, tail_hbm, out_hbm, idx_v, src_v, rows, stage,
           gsem, ssem):
        wid = lax.axis_index("c") * NS + lax.axis_index("s")
        iota = lax.iota(jnp.int32, 16)

        pltpu.sync_copy(tok_hbm.at[:, wid], idx_v)
        for b in range(2):
            pltpu.sync_copy(tail_hbm, rows.at[b, pl.ds(128, 64)])

        @pl.loop(0, SB)
        def _(sb):
            for sl in range(8):
                for b0 in range(8):
                    sl_ds = pl.ds(b0 * 16, 16)
                    ids = idx_v.at[sb, sl, sl_ds][...]
                    big = ids >= TAIL0
                    pos = iota + (b0 * 16)
                    src_v.at[sb, sl, sl_ds][...] = jnp.where(
                        big, ids - (TAIL0 - 128), pos)
                    idx_v.at[sb, sl, sl_ds][...] = jnp.where(big, 0, ids)

        def gather(s, b):
            return pltpu.make_async_copy(
                tab_hbm.at[idx_v.at[lax.shift_right_logical(s, 3),
                                    lax.bitwise_and(s, 7)]],
                rows.at[b, pl.ds(0, 128)], gsem.at[b])

        def stores(s, b):
            return [
                pltpu.make_async_copy(
                    stage.at[b, cg], out_hbm.at[s, cg, wid], ssem.at[b])
                for cg in range(8)
            ]

        def transpose_scale(s, b):
            r_ref = rows.at[b]
            s_ref = stage.at[b]
            sb = lax.shift_right_logical(s, 3)
            sl = lax.bitwise_and(s, 7)
            for b0 in range(8):
                tok = src_v.at[sb, sl, pl.ds(b0 * 16, 16)][...]

                @pl.loop(0, 8)
                def _(cl, tok=tok, b0=b0):
                    for cg in range(8):
                        col = jnp.broadcast_to(
                            (cg * 8 + cl).astype(jnp.int32), (16,))
                        vec = plsc.load_gather(r_ref, [tok, col])
                        s_ref.at[cg, cl, pl.ds(b0 * 16, 16)][...] = (
                            vec * SCALE)

        def process(s, b, wait_store, issue_next):
            gather(s, b).wait()
            if issue_next:
                gather(s + 1, 1 - b).start()
            if wait_store:
                for cp in stores(s, b):
                    cp.wait()
            transpose_scale(s, b)
            for cp in stores(s, b):
                cp.start()

        gather(0, 0).start()
        process(0, 0, wait_store=False, issue_next=True)
        process(1, 1, wait_store=False, issue_next=True)

        @pl.loop(1, 99)
        def _(g):
            s0 = 2 * g
            process(s0, 0, wait_store=True, issue_next=True)
            process(s0 + 1, 1, wait_store=True, issue_next=True)

        process(198, 0, wait_store=True, issue_next=True)
        process(199, 1, wait_store=True, issue_next=False)
        for b, s in ((0, 198), (1, 199)):
            for cp in stores(s, b):
                cp.wait()

    return k2(tok6, tab_rows, tail)


def kernel(token_ids, embedding_table):
    bsz, seq = token_ids.shape
    tab2 = _table_format(embedding_table.T)
    tab_rows = tab2.reshape(V, D)
    tail = embedding_table[TAIL0:, :]
    tok6 = (token_ids.astype(jnp.int32).T
            .reshape(SB, 8, BB, 128).transpose(0, 2, 1, 3))
    out5 = _gather_scale(tok6, tab_rows, tail)
    return out5.transpose(2, 4, 0, 1, 3).reshape(bsz, seq, D)


# speedup vs baseline: 1.9647x; 1.9647x over previous
import math

import jax
import jax.numpy as jnp
from jax import lax
from jax.experimental import pallas as pl
from jax.experimental.pallas import tpu as pltpu
from jax.experimental.pallas import tpu_sc as plsc

def _cp(tc_tiling):
    cp = pltpu.CompilerParams(use_tc_tiling_on_sc=tc_tiling)
    if "needs_layout_passes" in pltpu.CompilerParams.__dataclass_fields__:
        import dataclasses
        cp = dataclasses.replace(cp, needs_layout_passes=False)
    return cp


NC = 2
NS = 16
NW = NC * NS
V = 1000000
D = 64
SCALE = math.sqrt(D)

K0_CHUNKS_MAIN = 7808
K0_PER_W = K0_CHUNKS_MAIN // NW


def _table_format(t_cm):
    mesh = plsc.VectorSubcoreMesh(core_axis_name="c", subcore_axis_name="s")

    @pl.kernel(
        out_type=jax.ShapeDtypeStruct((V // 2, 128), jnp.float32),
        mesh=mesh,
        compiler_params=_cp(True),
        scratch_types=[
            pltpu.VMEM((2, 64, 128), jnp.float32),
            pltpu.VMEM((2, 64, 128), jnp.float32),
            pltpu.SemaphoreType.DMA((2,)),
            pltpu.SemaphoreType.DMA((2,)),
        ],
    )
    def k0(tcm_hbm, out_hbm, vbuf, sbuf, gsem, ssem):
        wid = lax.axis_index("c") * NS + lax.axis_index("s")
        j0 = wid * K0_PER_W
        iota = lax.iota(jnp.int32, 16)

        def in_copies(j, b):
            return [
                pltpu.make_async_copy(
                    tcm_hbm.at[pl.ds(cg * 8, 8), pl.ds(j * 128, 128)],
                    vbuf.at[b, pl.ds(cg * 8, 8), :], gsem.at[b])
                for cg in range(8)
            ]

        def store_copy(j, b, rows=64):
            return pltpu.make_async_copy(
                sbuf.at[b, pl.ds(0, rows), :],
                out_hbm.at[pl.ds(j * 64, rows), :], ssem.at[b])

        def transpose_chunk(b, nrow=64):
            v = vbuf.at[b]
            s = sbuf.at[b]

            @plsc.parallel_loop(0, 2 * nrow, unroll=8)
            def _(q):
                r = jnp.broadcast_to(q.astype(jnp.int32), (16,))
                p = lax.shift_right_logical(q, 1)
                off = lax.bitwise_and(q, 1) * 64
                for l0 in range(4):
                    c = iota + (l0 * 16)
                    vec = plsc.load_gather(v, [c, r])
                    s.at[p, pl.ds(off + l0 * 16, 16)][...] = vec

        def process(j, b, wait_store, issue_next):
            for cp in in_copies(j, b):
                cp.wait()
            if wait_store:
                store_copy(j, b).wait()
            transpose_chunk(b)
            if issue_next:
                for cp in in_copies(j + 2, b):
                    cp.start()
            store_copy(j, b).start()

        for b in range(2):
            for cp in in_copies(j0 + b, b):
                cp.start()
        for b in range(2):
            process(j0 + b, b, wait_store=False, issue_next=True)

        @pl.loop(1, K0_PER_W // 2 - 1)
        def _(g):
            jj = j0 + 2 * g
            for b in range(2):
                process(jj + b, b, wait_store=True, issue_next=True)

        jj = j0 + K0_PER_W - 2
        for b in range(2):
            process(jj + b, b, wait_store=True, issue_next=False)
        for b in range(2):
            store_copy(jj + b, b).wait()

        @pl.when(wid < 4)
        def _():
            j = K0_CHUNKS_MAIN + wid
            for cp in in_copies(j, 0):
                cp.start()
            for cp in in_copies(j, 0):
                cp.wait()
            transpose_chunk(0)
            store_copy(j, 0).start()
            store_copy(j, 0).wait()

    return k0(t_cm)


SB = 25
BB = 32


TAIL0 = K0_CHUNKS_MAIN * 128 + 4 * 128


def _gather_scale(tok6, tab_rows, tail):
    mesh = plsc.VectorSubcoreMesh(core_axis_name="c", subcore_axis_name="s")

    @pl.kernel(
        out_type=jax.ShapeDtypeStruct((200, 8, BB, 8, 128), jnp.float32),
        mesh=mesh,
        compiler_params=_cp(False),
        scratch_types=[
            pltpu.VMEM((SB, 8, 128), jnp.int32),
            pltpu.VMEM((SB, 8, 128), jnp.int32),
            pltpu.VMEM((2, 192, D), jnp.float32),
            pltpu.VMEM((2, 8, 8, 128), jnp.float32),
            pltpu.SemaphoreType.DMA((2,)),
            pltpu.SemaphoreType.DMA((2,)),
        ],
    )
    def k2(tok_hbm, tab_hbm, tail_hbm, out_hbm, idx_v, src_v, rows, stage,
           gsem, ssem):
        wid = lax.axis_index("c") * NS + lax.axis_index("s")
        iota = lax.iota(jnp.int32, 16)

        pltpu.sync_copy(tok_hbm.at[:, wid], idx_v)
        for b in range(2):
            pltpu.sync_copy(tail_hbm, rows.at[b, pl.ds(128, 64)])

        @pl.loop(0, SB)
        def _(sb):
            for sl in range(8):
                for b0 in range(8):
                    sl_ds = pl.ds(b0 * 16, 16)
                    ids = idx_v.at[sb, sl, sl_ds][...]
                    big = ids >= TAIL0
                    pos = iota + (b0 * 16)
                    src_v.at[sb, sl, sl_ds][...] = jnp.where(
                        big, ids - (TAIL0 - 128), pos)
                    idx_v.at[sb, sl, sl_ds][...] = jnp.where(big, 0, ids)

        def gather(s, b):
            return pltpu.make_async_copy(
                tab_hbm.at[idx_v.at[lax.shift_right_logical(s, 3),
                                    lax.bitwise_and(s, 7)]],
                rows.at[b, pl.ds(0, 128)], gsem.at[b])

        def stores(s, b):
            return [
                pltpu.make_async_copy(
                    stage.at[b, cg], out_hbm.at[s, cg, wid], ssem.at[b])
                for cg in range(8)
            ]

        def transpose_scale(s, b):
            r_ref = rows.at[b]
            s_ref = stage.at[b]
            sb = lax.shift_right_logical(s, 3)
            sl = lax.bitwise_and(s, 7)
            toks = [src_v.at[sb, sl, pl.ds(b0 * 16, 16)][...]
                    for b0 in range(8)]

            @plsc.parallel_loop(0, 64, unroll=4)
            def _(c):
                col = jnp.broadcast_to(c.astype(jnp.int32), (16,))
                cg = lax.shift_right_logical(c, 3)
                cl = lax.bitwise_and(c, 7)
                for b0 in range(8):
                    vec = plsc.load_gather(r_ref, [toks[b0], col])
                    s_ref.at[cg, cl, pl.ds(b0 * 16, 16)][...] = vec * SCALE

        def process(s, b, wait_store, issue_next):
            gather(s, b).wait()
            if issue_next:
                gather(s + 1, 1 - b).start()
            if wait_store:
                for cp in stores(s, b):
                    cp.wait()
            transpose_scale(s, b)
            for cp in stores(s, b):
                cp.start()

        gather(0, 0).start()
        process(0, 0, wait_store=False, issue_next=True)
        process(1, 1, wait_store=False, issue_next=True)

        @pl.loop(1, 99)
        def _(g):
            s0 = 2 * g
            process(s0, 0, wait_store=True, issue_next=True)
            process(s0 + 1, 1, wait_store=True, issue_next=True)

        process(198, 0, wait_store=True, issue_next=True)
        process(199, 1, wait_store=True, issue_next=False)
        for b, s in ((0, 198), (1, 199)):
            for cp in stores(s, b):
                cp.wait()

    return k2(tok6, tab_rows, tail)


def kernel(token_ids, embedding_table):
    bsz, seq = token_ids.shape
    tab2 = _table_format(embedding_table.T)
    tab_rows = tab2.reshape(V, D)
    tail = embedding_table[TAIL0:, :]
    tok6 = (token_ids.astype(jnp.int32).T
            .reshape(SB, 8, BB, 128).transpose(0, 2, 1, 3))
    out5 = _gather_scale(tok6, tab_rows, tail)
    return out5.transpose(2, 4, 0, 1, 3).reshape(bsz, seq, D)
